# Initial kernel scaffold; baseline (speedup 1.0000x reference)
import jax
import jax.numpy as jnp
from jax.experimental import pallas as pl


def kernel(x, weight):
    raise NotImplementedError("write your pallas kernel here")


# speedup vs baseline: 1.1016x; 1.1016x over previous
import functools

import jax
import jax.numpy as jnp
from jax import lax
from jax.experimental import pallas as pl
from jax.experimental.pallas import tpu as pltpu, tpu_sc as plsc

NUM_ROWS = 4096 * 26
DIM = 64
CHUNK = 128
NC, NS = 2, 16
NW = NC * NS
ROWS_PER_W = NUM_ROWS // NW
CHUNKS_PER_W = ROWS_PER_W // CHUNK


IDX_WIN = 32


def _emb_body(idx_hbm, table_hbm, out_hbm, idx_v, rows_v, sem):
    wid = lax.axis_index("s") * NC + lax.axis_index("c")
    start = wid * CHUNKS_PER_W
    r = lax.rem(start, 8)
    a = pl.multiple_of(start - r, 8)
    pltpu.sync_copy(idx_hbm.at[pl.ds(a, IDX_WIN)], idx_v)
    base = wid * ROWS_PER_W

    def step(j, carry):
        pltpu.async_copy(table_hbm.at[idx_v.at[r + j]], rows_v, sem).wait()
        pltpu.sync_copy(rows_v, out_hbm.at[pl.ds(base + j * CHUNK, CHUNK)])
        return carry

    lax.fori_loop(0, CHUNKS_PER_W, step, 0)


@jax.jit
def _embedding_sc(idx2d, weight):
    mesh = plsc.VectorSubcoreMesh(core_axis_name="c", subcore_axis_name="s")
    f = pl.kernel(
        _emb_body,
        out_type=jax.ShapeDtypeStruct((NUM_ROWS, DIM), jnp.float32),
        mesh=mesh,
        scratch_types=[
            pltpu.VMEM((IDX_WIN, CHUNK), jnp.int32),
            pltpu.VMEM((CHUNK, DIM), jnp.float32),
            pltpu.SemaphoreType.DMA,
        ],
        compiler_params=pltpu.CompilerParams(use_tc_tiling_on_sc=False),
    )
    return f(idx2d, weight)


def kernel(x, weight):
    idx2d = x.reshape(NUM_ROWS // CHUNK, CHUNK).astype(jnp.int32)
    out = _embedding_sc(idx2d, weight)
    return out.reshape(x.shape[0], x.shape[1], DIM)


# speedup vs baseline: 1.2121x; 1.1003x over previous
import functools

import jax
import jax.numpy as jnp
from jax import lax
from jax.experimental import pallas as pl
from jax.experimental.pallas import tpu as pltpu, tpu_sc as plsc

NUM_ROWS = 4096 * 26
DIM = 64
CHUNK = 128
NC, NS = 2, 16
NW = NC * NS
ROWS_PER_W = NUM_ROWS // NW
CHUNKS_PER_W = ROWS_PER_W // CHUNK


IDX_WIN = 32
NB = 4
DEPTH = 2


def _emb_body(idx_hbm, table_hbm, out_hbm, idx_v, rows_v,
              gs0, gs1, gs2, gs3, os0, os1, os2, os3):
    gs = (gs0, gs1, gs2, gs3)
    os_ = (os0, os1, os2, os3)
    wid = lax.axis_index("s") * NC + lax.axis_index("c")
    start = wid * CHUNKS_PER_W
    r = lax.rem(start, 8)
    a = pl.multiple_of(start - r, 8)
    pltpu.sync_copy(idx_hbm.at[pl.ds(a, IDX_WIN)], idx_v)
    base = wid * ROWS_PER_W

    gd, od = {}, {}
    for j in range(CHUNKS_PER_W + DEPTH):
        if j < CHUNKS_PER_W:
            b = j % NB
            if j >= NB:
                od[j - NB].wait()
            gd[j] = pltpu.async_copy(
                table_hbm.at[idx_v.at[r + j]], rows_v.at[b], gs[b])
        k = j - DEPTH
        if k >= 0:
            gd[k].wait()
            od[k] = pltpu.async_copy(
                rows_v.at[k % NB],
                out_hbm.at[pl.ds(base + k * CHUNK, CHUNK)],
                os_[k % NB])
    for k in range(CHUNKS_PER_W - NB, CHUNKS_PER_W):
        od[k].wait()


@jax.jit
def _embedding_sc(idx2d, weight):
    mesh = plsc.VectorSubcoreMesh(core_axis_name="c", subcore_axis_name="s")
    f = pl.kernel(
        _emb_body,
        out_type=jax.ShapeDtypeStruct((NUM_ROWS, DIM), jnp.float32),
        mesh=mesh,
        scratch_types=[
            pltpu.VMEM((IDX_WIN, CHUNK), jnp.int32),
            pltpu.VMEM((NB, CHUNK, DIM), jnp.float32),
        ] + [pltpu.SemaphoreType.DMA] * (2 * NB),
        compiler_params=pltpu.CompilerParams(use_tc_tiling_on_sc=False),
    )
    return f(idx2d, weight)


def kernel(x, weight):
    idx2d = x.reshape(NUM_ROWS // CHUNK, CHUNK).astype(jnp.int32)
    out = _embedding_sc(idx2d, weight)
    return out.reshape(x.shape[0], x.shape[1], DIM)


# speedup vs baseline: 1.2184x; 1.0052x over previous
import jax
import jax.numpy as jnp
from jax import lax
from jax.experimental import pallas as pl
from jax.experimental.pallas import tpu as pltpu, tpu_sc as plsc

NUM_ROWS = 4096 * 26
DIM = 64
NC, NS = 2, 16
NW = NC * NS
ROWS_PER_W = NUM_ROWS // NW
CG = 832
NCHUNK = ROWS_PER_W // CG
NB = 2


def _emb_body(idx_hbm, table_hbm, out_hbm, idx_v, rows_v, gs0, gs1, os0, os1):
    gs = (gs0, gs1)
    os_ = (os0, os1)
    wid = lax.axis_index("s") * NC + lax.axis_index("c")
    base = wid * ROWS_PER_W
    pltpu.sync_copy(idx_hbm.at[pl.ds(base, ROWS_PER_W)], idx_v)

    gd, od = {}, {}
    for j in range(NCHUNK + 1):
        if j < NCHUNK:
            b = j % NB
            if j >= NB:
                od[j - NB].wait()
            gd[j] = pltpu.async_copy(
                table_hbm.at[idx_v.at[pl.ds(j * CG, CG)]], rows_v.at[b], gs[b])
        k = j - 1
        if k >= 0:
            gd[k].wait()
            od[k] = pltpu.async_copy(
                rows_v.at[k % NB],
                out_hbm.at[pl.ds(base + k * CG, CG)],
                os_[k % NB])
    for k in range(NCHUNK - NB, NCHUNK):
        od[k].wait()


@jax.jit
def _embedding_sc(idx, weight):
    mesh = plsc.VectorSubcoreMesh(core_axis_name="c", subcore_axis_name="s")
    f = pl.kernel(
        _emb_body,
        out_type=jax.ShapeDtypeStruct((NUM_ROWS, DIM), jnp.float32),
        mesh=mesh,
        scratch_types=[
            pltpu.VMEM((ROWS_PER_W,), jnp.int32),
            pltpu.VMEM((NB, CG, DIM), jnp.float32),
        ] + [pltpu.SemaphoreType.DMA] * (2 * NB),
        compiler_params=pltpu.CompilerParams(use_tc_tiling_on_sc=False),
    )
    return f(idx, weight)


def kernel(x, weight):
    idx = x.reshape(NUM_ROWS).astype(jnp.int32)
    out = _embedding_sc(idx, weight)
    return out.reshape(x.shape[0], x.shape[1], DIM)
